# baseline (speedup 1.0000x reference)
import jax
import jax.numpy as jnp
from jax.experimental import pallas as pl
from jax.experimental.pallas import tpu as pltpu

N_NODES = 10000
N_EDGES = 320000
D_FEAT = 128
D_EDGE = 16
D_GLOBAL = 128

NUM_BLOCKS = 10
E_STREAMS = 10
N_STREAMS = 5
BE = N_EDGES // (E_STREAMS * NUM_BLOCKS)
BN = N_NODES // (N_STREAMS * NUM_BLOCKS)


def _body(*refs):
    (edge_refs, node_refs, (global_ref, w1_ref, b1_ref, w2_ref, b2_ref),
     (out_ref,), (acc_e_ref, acc_n_ref)) = (
        refs[0:E_STREAMS], refs[E_STREAMS:E_STREAMS + N_STREAMS],
        refs[E_STREAMS + N_STREAMS:E_STREAMS + N_STREAMS + 5],
        refs[E_STREAMS + N_STREAMS + 5:E_STREAMS + N_STREAMS + 6],
        refs[E_STREAMS + N_STREAMS + 6:])
    i = pl.program_id(0)

    @pl.when(i == 0)
    def _init():
        acc_e_ref[...] = jnp.zeros_like(acc_e_ref)
        acc_n_ref[...] = jnp.zeros_like(acc_n_ref)

    acc = acc_e_ref[...]
    for e_ref in edge_refs:
        e = e_ref[...]
        for k in range(BE // 128):
            acc = acc + e[:, k * 128:(k + 1) * 128]
    acc_e_ref[...] = acc
    nsum = jnp.sum(node_refs[0][...], axis=0, keepdims=True)
    for n_ref in node_refs[1:]:
        nsum = nsum + jnp.sum(n_ref[...], axis=0, keepdims=True)
    acc_n_ref[...] += nsum

    @pl.when(i == NUM_BLOCKS - 1)
    def _finish():
        esum = jnp.sum(acc_e_ref[...], axis=1, keepdims=True)
        agg_n = acc_n_ref[...] * (1.0 / N_NODES)
        g = global_ref[...]
        w1 = w1_ref[...]
        h_e = jax.lax.dot_general(
            esum * (1.0 / N_EDGES), w1[D_GLOBAL:D_GLOBAL + D_EDGE, :],
            (((0,), (0,)), ((), ())),
            preferred_element_type=jnp.float32,
        )
        pre = (
            jnp.dot(g, w1[0:D_GLOBAL, :], preferred_element_type=jnp.float32)
            + h_e
            + jnp.dot(agg_n, w1[D_GLOBAL + D_EDGE:, :],
                      preferred_element_type=jnp.float32)
            + b1_ref[...]
        )
        h = jnp.maximum(pre, 0.0)
        out_ref[...] = (
            jnp.dot(h, w2_ref[...], preferred_element_type=jnp.float32)
            + b2_ref[...]
        )


def kernel(node_attr, edge_index, edge_attr, global_attr, W1, b1, W2, b2):
    del edge_index
    b1_2d = b1.reshape(1, -1)
    b2_2d = b2.reshape(1, -1)
    edge_t = edge_attr.T
    return pl.pallas_call(
        _body,
        grid=(NUM_BLOCKS,),
        in_specs=[
            *[pl.BlockSpec((D_EDGE, BE),
                           (lambda j: lambda i: (0, i + j * NUM_BLOCKS))(j))
              for j in range(E_STREAMS)],
            *[pl.BlockSpec((BN, D_FEAT),
                           (lambda j: lambda i: (i + j * NUM_BLOCKS, 0))(j))
              for j in range(N_STREAMS)],
            pl.BlockSpec((1, D_GLOBAL), lambda i: (0, 0)),
            pl.BlockSpec((D_GLOBAL + D_EDGE + D_FEAT, 32), lambda i: (0, 0)),
            pl.BlockSpec((1, 32), lambda i: (0, 0)),
            pl.BlockSpec((32, D_FEAT), lambda i: (0, 0)),
            pl.BlockSpec((1, D_FEAT), lambda i: (0, 0)),
        ],
        out_specs=pl.BlockSpec((1, D_FEAT), lambda i: (0, 0)),
        out_shape=jax.ShapeDtypeStruct((1, D_FEAT), jnp.float32),
        scratch_shapes=[
            pltpu.VMEM((D_EDGE, 128), jnp.float32),
            pltpu.VMEM((1, D_FEAT), jnp.float32),
        ],
    )(*([edge_t] * E_STREAMS), *([node_attr] * N_STREAMS), global_attr,
      W1, b1_2d, W2, b2_2d)


# speedup vs baseline: 1.1352x; 1.1352x over previous
import jax
import jax.numpy as jnp
from jax.experimental import pallas as pl
from jax.experimental.pallas import tpu as pltpu

N_NODES = 10000
N_EDGES = 320000
D_FEAT = 128
D_EDGE = 16
D_GLOBAL = 128

NUM_BLOCKS = 2
E_STREAMS = 10
N_STREAMS = 5
BE = N_EDGES // (E_STREAMS * NUM_BLOCKS)
BN = N_NODES // (N_STREAMS * NUM_BLOCKS)


def _body(*refs):
    (edge_refs, node_refs, (global_ref, w1_ref, b1_ref, w2_ref, b2_ref),
     (out_ref,), (acc_e_ref, acc_n_ref)) = (
        refs[0:E_STREAMS], refs[E_STREAMS:E_STREAMS + N_STREAMS],
        refs[E_STREAMS + N_STREAMS:E_STREAMS + N_STREAMS + 5],
        refs[E_STREAMS + N_STREAMS + 5:E_STREAMS + N_STREAMS + 6],
        refs[E_STREAMS + N_STREAMS + 6:])
    i = pl.program_id(0)

    @pl.when(i == 0)
    def _init():
        acc_e_ref[...] = jnp.zeros_like(acc_e_ref)
        acc_n_ref[...] = jnp.zeros_like(acc_n_ref)

    acc = acc_e_ref[...]
    for e_ref in edge_refs:
        e = e_ref[...]
        for k in range(BE // 128):
            acc = acc + e[:, k * 128:(k + 1) * 128]
    acc_e_ref[...] = acc
    nsum = jnp.sum(node_refs[0][...], axis=0, keepdims=True)
    for n_ref in node_refs[1:]:
        nsum = nsum + jnp.sum(n_ref[...], axis=0, keepdims=True)
    acc_n_ref[...] += nsum

    @pl.when(i == NUM_BLOCKS - 1)
    def _finish():
        esum = jnp.sum(acc_e_ref[...], axis=1, keepdims=True)
        agg_n = acc_n_ref[...] * (1.0 / N_NODES)
        g = global_ref[...]
        w1 = w1_ref[...]
        h_e = jax.lax.dot_general(
            esum * (1.0 / N_EDGES), w1[D_GLOBAL:D_GLOBAL + D_EDGE, :],
            (((0,), (0,)), ((), ())),
            preferred_element_type=jnp.float32,
        )
        pre = (
            jnp.dot(g, w1[0:D_GLOBAL, :], preferred_element_type=jnp.float32)
            + h_e
            + jnp.dot(agg_n, w1[D_GLOBAL + D_EDGE:, :],
                      preferred_element_type=jnp.float32)
            + b1_ref[...]
        )
        h = jnp.maximum(pre, 0.0)
        out_ref[...] = (
            jnp.dot(h, w2_ref[...], preferred_element_type=jnp.float32)
            + b2_ref[...]
        )


def kernel(node_attr, edge_index, edge_attr, global_attr, W1, b1, W2, b2):
    del edge_index
    b1_2d = b1.reshape(1, -1)
    b2_2d = b2.reshape(1, -1)
    edge_t = edge_attr.T
    return pl.pallas_call(
        _body,
        grid=(NUM_BLOCKS,),
        in_specs=[
            *[pl.BlockSpec((D_EDGE, BE),
                           (lambda j: lambda i: (0, i + j * NUM_BLOCKS))(j))
              for j in range(E_STREAMS)],
            *[pl.BlockSpec((BN, D_FEAT),
                           (lambda j: lambda i: (i + j * NUM_BLOCKS, 0))(j))
              for j in range(N_STREAMS)],
            pl.BlockSpec((1, D_GLOBAL), lambda i: (0, 0)),
            pl.BlockSpec((D_GLOBAL + D_EDGE + D_FEAT, 32), lambda i: (0, 0)),
            pl.BlockSpec((1, 32), lambda i: (0, 0)),
            pl.BlockSpec((32, D_FEAT), lambda i: (0, 0)),
            pl.BlockSpec((1, D_FEAT), lambda i: (0, 0)),
        ],
        out_specs=pl.BlockSpec((1, D_FEAT), lambda i: (0, 0)),
        out_shape=jax.ShapeDtypeStruct((1, D_FEAT), jnp.float32),
        scratch_shapes=[
            pltpu.VMEM((D_EDGE, 128), jnp.float32),
            pltpu.VMEM((1, D_FEAT), jnp.float32),
        ],
    )(*([edge_t] * E_STREAMS), *([node_attr] * N_STREAMS), global_attr,
      W1, b1_2d, W2, b2_2d)


# speedup vs baseline: 1.1836x; 1.0426x over previous
import jax
import jax.numpy as jnp
from jax.experimental import pallas as pl
from jax.experimental.pallas import tpu as pltpu

N_NODES = 10000
N_EDGES = 320000
D_FEAT = 128
D_EDGE = 16
D_GLOBAL = 128

NUM_BLOCKS = 5
E_STREAMS = 10
N_STREAMS = 5
BE = N_EDGES // (E_STREAMS * NUM_BLOCKS)
BN = N_NODES // (N_STREAMS * NUM_BLOCKS)


def _body(*refs):
    (edge_refs, node_refs, (global_ref, w1_ref, b1_ref, w2_ref, b2_ref),
     (out_ref,), (acc_e_ref, acc_n_ref)) = (
        refs[0:E_STREAMS], refs[E_STREAMS:E_STREAMS + N_STREAMS],
        refs[E_STREAMS + N_STREAMS:E_STREAMS + N_STREAMS + 5],
        refs[E_STREAMS + N_STREAMS + 5:E_STREAMS + N_STREAMS + 6],
        refs[E_STREAMS + N_STREAMS + 6:])
    i = pl.program_id(0)

    @pl.when(i == 0)
    def _init():
        acc_e_ref[...] = jnp.zeros_like(acc_e_ref)
        acc_n_ref[...] = jnp.zeros_like(acc_n_ref)

    acc = acc_e_ref[...]
    for e_ref in edge_refs:
        e = e_ref[...]
        for k in range(BE // 128):
            acc = acc + e[:, k * 128:(k + 1) * 128]
    acc_e_ref[...] = acc
    nsum = jnp.sum(node_refs[0][...], axis=0, keepdims=True)
    for n_ref in node_refs[1:]:
        nsum = nsum + jnp.sum(n_ref[...], axis=0, keepdims=True)
    acc_n_ref[...] += nsum

    @pl.when(i == NUM_BLOCKS - 1)
    def _finish():
        esum = jnp.sum(acc_e_ref[...], axis=1, keepdims=True)
        agg_n = acc_n_ref[...] * (1.0 / N_NODES)
        g = global_ref[...]
        w1 = w1_ref[...]
        h_e = jax.lax.dot_general(
            esum * (1.0 / N_EDGES), w1[D_GLOBAL:D_GLOBAL + D_EDGE, :],
            (((0,), (0,)), ((), ())),
            preferred_element_type=jnp.float32,
        )
        pre = (
            jnp.dot(g, w1[0:D_GLOBAL, :], preferred_element_type=jnp.float32)
            + h_e
            + jnp.dot(agg_n, w1[D_GLOBAL + D_EDGE:, :],
                      preferred_element_type=jnp.float32)
            + b1_ref[...]
        )
        h = jnp.maximum(pre, 0.0)
        out_ref[...] = (
            jnp.dot(h, w2_ref[...], preferred_element_type=jnp.float32)
            + b2_ref[...]
        )


def kernel(node_attr, edge_index, edge_attr, global_attr, W1, b1, W2, b2):
    del edge_index
    b1_2d = b1.reshape(1, -1)
    b2_2d = b2.reshape(1, -1)
    edge_t = edge_attr.T
    return pl.pallas_call(
        _body,
        grid=(NUM_BLOCKS,),
        in_specs=[
            *[pl.BlockSpec((D_EDGE, BE),
                           (lambda j: lambda i: (0, i + j * NUM_BLOCKS))(j))
              for j in range(E_STREAMS)],
            *[pl.BlockSpec((BN, D_FEAT),
                           (lambda j: lambda i: (i + j * NUM_BLOCKS, 0))(j))
              for j in range(N_STREAMS)],
            pl.BlockSpec((1, D_GLOBAL), lambda i: (0, 0)),
            pl.BlockSpec((D_GLOBAL + D_EDGE + D_FEAT, 32), lambda i: (0, 0)),
            pl.BlockSpec((1, 32), lambda i: (0, 0)),
            pl.BlockSpec((32, D_FEAT), lambda i: (0, 0)),
            pl.BlockSpec((1, D_FEAT), lambda i: (0, 0)),
        ],
        out_specs=pl.BlockSpec((1, D_FEAT), lambda i: (0, 0)),
        out_shape=jax.ShapeDtypeStruct((1, D_FEAT), jnp.float32),
        scratch_shapes=[
            pltpu.VMEM((D_EDGE, 128), jnp.float32),
            pltpu.VMEM((1, D_FEAT), jnp.float32),
        ],
    )(*([edge_t] * E_STREAMS), *([node_attr] * N_STREAMS), global_attr,
      W1, b1_2d, W2, b2_2d)
